# baseline (speedup 1.0000x reference)
import jax
import jax.numpy as jnp
from jax import lax
from jax.experimental import pallas as pl
from jax.experimental.pallas import tpu as pltpu
from jax.experimental.pallas import tpu_sc as plsc

B = 16384
D = 64
NC, NS = 2, 16
NW = NC * NS
BPW = B // NW
CH = 128
NCH = BPW // CH
L = 16


def _body(ux_hbm, ix_hbm, uid_hbm, iid_hbm, out_hbm,
          idx_u, idx_i, rows_u, rows_i, out_v, sem):
    wid = lax.axis_index("s") * NC + lax.axis_index("c")
    base = wid * BPW

    for c in range(NCH):
        pltpu.sync_copy(ux_hbm.at[pl.ds(base + c * CH, CH)], idx_u.at[c])
        pltpu.sync_copy(ix_hbm.at[pl.ds(base + c * CH, CH)], idx_i.at[c])

    copies = []
    for c in range(NCH):
        copies.append(pltpu.async_copy(
            uid_hbm.at[idx_u.at[c]], rows_u.at[pl.ds(c * CH, CH)], sem))
        copies.append(pltpu.async_copy(
            iid_hbm.at[idx_i.at[c]], rows_i.at[pl.ds(c * CH, CH)], sem))
    for cp in copies:
        cp.wait()

    lanes = lax.iota(jnp.int32, L)

    def blk(b, carry):
        r0 = b * L
        row_idx = r0 + lanes
        accs = [jnp.zeros((L,), jnp.float32) for _ in range(4)]
        for d in range(D):
            col = jnp.full((L,), d, jnp.int32)
            u = plsc.load_gather(rows_u, [row_idx, col])
            v = plsc.load_gather(rows_i, [row_idx, col])
            accs[d % 4] = accs[d % 4] + u * v
        out_v[pl.ds(r0, L)] = (accs[0] + accs[1]) + (accs[2] + accs[3])
        return carry

    lax.fori_loop(0, BPW // L, blk, 0)
    pltpu.sync_copy(out_v, out_hbm.at[pl.ds(base, BPW)])


def kernel(x, uid_table, iid_table):
    ux = x[:, 0].astype(jnp.int32)
    ix = x[:, 1].astype(jnp.int32)
    mesh = plsc.VectorSubcoreMesh(
        core_axis_name="c", subcore_axis_name="s",
        num_cores=NC, num_subcores=NS)
    run = pl.kernel(
        _body,
        out_type=jax.ShapeDtypeStruct((B,), jnp.float32),
        mesh=mesh,
        compiler_params=pltpu.CompilerParams(
            needs_layout_passes=False, use_tc_tiling_on_sc=False),
        scratch_types=[
            pltpu.VMEM((NCH, CH), jnp.int32),
            pltpu.VMEM((NCH, CH), jnp.int32),
            pltpu.VMEM((BPW, D), jnp.float32),
            pltpu.VMEM((BPW, D), jnp.float32),
            pltpu.VMEM((BPW,), jnp.float32),
            pltpu.SemaphoreType.DMA,
        ],
    )
    return run(ux, ix, uid_table, iid_table)
